# baseline (speedup 1.0000x reference)
import functools

import jax
import jax.numpy as jnp
from jax.experimental import pallas as pl
from jax.experimental.pallas import tpu as pltpu

_P = jax.sharding.PartitionSpec


def _body(x_ref, adj_ref, w_ref, mlp_w_ref, mlp_b_ref, out_ref, xw_ref):
    @pl.when(pl.program_id(0) == 0)
    def _():
        xw_ref[...] = jnp.dot(x_ref[...], w_ref[...],
                              preferred_element_type=jnp.float32)

    a = jnp.dot(adj_ref[...], xw_ref[...], preferred_element_type=jnp.float32)
    a = jnp.maximum(a, 0.0)
    y = jax.lax.dot_general(a, mlp_w_ref[...],
                            dimension_numbers=(((1,), (1,)), ((), ())),
                            preferred_element_type=jnp.float32)
    y = y + mlp_b_ref[...]
    zmax = jnp.max(y, axis=1, keepdims=True)
    zmin = jnp.min(y, axis=1, keepdims=True)
    z = (y - zmin) / (zmax - zmin)
    nrm = jnp.sqrt(jnp.sum(z * z, axis=1, keepdims=True))
    out_ref[...] = z / jnp.maximum(nrm, 1e-12)


def _spmm_fused(x, adj, W, mlp_w, mlp_b2, bm):
    m, n = adj.shape
    d_in = x.shape[1]
    d_hid = W.shape[1]
    d_out = mlp_w.shape[0]
    return pl.pallas_call(
        _body,
        grid=(m // bm,),
        in_specs=[
            pl.BlockSpec((n, d_in), lambda i: (0, 0)),
            pl.BlockSpec((bm, n), lambda i: (i, 0)),
            pl.BlockSpec((d_in, d_hid), lambda i: (0, 0)),
            pl.BlockSpec((d_out, d_hid), lambda i: (0, 0)),
            pl.BlockSpec((1, d_out), lambda i: (0, 0)),
        ],
        out_specs=pl.BlockSpec((bm, d_out), lambda i: (i, 0)),
        out_shape=jax.ShapeDtypeStruct((m, d_out), jnp.float32),
        scratch_shapes=[pltpu.VMEM((n, d_hid), jnp.float32)],
        compiler_params=pltpu.CompilerParams(
            dimension_semantics=("arbitrary",),
        ),
    )(x, adj, W, mlp_w, mlp_b2)


def _pick_bm(rows):
    return next(b for b in (400, 200, 80, 40, 8, 1) if rows % b == 0)


@functools.partial(jax.jit, static_argnames=("bm",))
def _run1(x, adj, W, mlp_w, mlp_b2, bm):
    return _spmm_fused(x, adj, W, mlp_w, mlp_b2, bm)


_SHARDED = {}


def _run_sharded(ndev):
    if ndev not in _SHARDED:
        mesh = jax.make_mesh((ndev,), ("i",))

        def per_shard(x, adj_shard, W, mlp_w, mlp_b2):
            return _spmm_fused(x, adj_shard, W, mlp_w, mlp_b2,
                               _pick_bm(adj_shard.shape[0]))

        fn = jax.jit(jax.shard_map(
            per_shard,
            mesh=mesh,
            in_specs=(_P(), _P("i", None), _P(), _P(), _P()),
            out_specs=_P("i", None),
            check_vma=False,
        ))
        row_shard = jax.NamedSharding(mesh, _P("i", None))
        repl = jax.NamedSharding(mesh, _P())
        _SHARDED[ndev] = (fn, row_shard, repl)
    return _SHARDED[ndev]


def kernel(input, adj, W, mlp_w, mlp_b):
    n = adj.shape[0]
    ndev = len(jax.devices())
    if ndev >= 2 and n % 2 == 0:
        fn, row_shard, repl = _run_sharded(2)
        args = (jax.device_put(input, repl), jax.device_put(adj, row_shard),
                jax.device_put(W, repl), jax.device_put(mlp_w, repl),
                jax.device_put(mlp_b.reshape(1, -1), repl))
        return fn(*args)
    return _run1(input, adj, W, mlp_w, mlp_b.reshape(1, -1), _pick_bm(n))


# speedup vs baseline: 5.7344x; 5.7344x over previous
import functools

import jax
import jax.numpy as jnp
from jax.experimental import pallas as pl
from jax.experimental.pallas import tpu as pltpu


def _body(x_ref, adj_ref, w_ref, mlp_w_ref, mlp_b_ref, out_ref, xw_ref):
    @pl.when(pl.program_id(0) == 0)
    def _():
        xw_ref[...] = jnp.dot(x_ref[...], w_ref[...],
                              preferred_element_type=jnp.float32)

    a = jnp.dot(adj_ref[...], xw_ref[...], preferred_element_type=jnp.float32)
    a = jnp.maximum(a, 0.0)
    y = jax.lax.dot_general(a, mlp_w_ref[...],
                            dimension_numbers=(((1,), (1,)), ((), ())),
                            preferred_element_type=jnp.float32)
    y = y + mlp_b_ref[...]
    zmax = jnp.max(y, axis=1, keepdims=True)
    zmin = jnp.min(y, axis=1, keepdims=True)
    z = (y - zmin) / (zmax - zmin)
    nrm = jnp.sqrt(jnp.sum(z * z, axis=1, keepdims=True))
    out_ref[...] = z / jnp.maximum(nrm, 1e-12)


def _spmm_fused(x, adj, W, mlp_w, mlp_b2, bm):
    m, n = adj.shape
    d_in = x.shape[1]
    d_hid = W.shape[1]
    d_out = mlp_w.shape[0]
    return pl.pallas_call(
        _body,
        grid=(m // bm,),
        in_specs=[
            pl.BlockSpec((n, d_in), lambda i: (0, 0)),
            pl.BlockSpec((bm, n), lambda i: (i, 0)),
            pl.BlockSpec((d_in, d_hid), lambda i: (0, 0)),
            pl.BlockSpec((d_out, d_hid), lambda i: (0, 0)),
            pl.BlockSpec((1, d_out), lambda i: (0, 0)),
        ],
        out_specs=pl.BlockSpec((bm, d_out), lambda i: (i, 0)),
        out_shape=jax.ShapeDtypeStruct((m, d_out), jnp.float32),
        scratch_shapes=[pltpu.VMEM((n, d_hid), jnp.float32)],
        compiler_params=pltpu.CompilerParams(
            dimension_semantics=("arbitrary",),
        ),
    )(x, adj, W, mlp_w, mlp_b2)


def _pick_bm(rows):
    return next(b for b in (400, 200, 80, 40, 8, 1) if rows % b == 0)


@functools.partial(jax.jit, static_argnames=("bm",))
def _run1(x, adj, W, mlp_w, mlp_b2, bm):
    return _spmm_fused(x, adj, W, mlp_w, mlp_b2, bm)


def kernel(input, adj, W, mlp_w, mlp_b):
    return _run1(input, adj, W, mlp_w, mlp_b.reshape(1, -1),
                 _pick_bm(adj.shape[0]))
